# baseline (speedup 1.0000x reference)
import jax
import jax.numpy as jnp
from jax.experimental import pallas as pl
from jax.experimental.pallas import tpu as pltpu

B, S, D_MODEL = 4, 8192, 1024
D_ROUTING = 256
N_INPUT = 32768
K = 2048
S_CHUNK = 1024
N_CHUNK = 4096


def _maxpool_body(x_ref, o_ref):
    s = pl.program_id(1)
    m = jnp.max(x_ref[0], axis=0, keepdims=True)[None]

    @pl.when(s == 0)
    def _init():
        o_ref[...] = m

    @pl.when(s != 0)
    def _acc():
        o_ref[...] = jnp.maximum(o_ref[...], m)


def _maxpool(x):
    out = pl.pallas_call(
        _maxpool_body,
        grid=(B, S // S_CHUNK),
        in_specs=[pl.BlockSpec((1, S_CHUNK, D_MODEL), lambda b, s: (b, s, 0))],
        out_specs=pl.BlockSpec((1, 1, D_MODEL), lambda b, s: (b, 0, 0)),
        out_shape=jax.ShapeDtypeStruct((B, 1, D_MODEL), jnp.float32),
        compiler_params=pltpu.CompilerParams(
            dimension_semantics=("parallel", "arbitrary")),
    )(x)
    return out.reshape(B, D_MODEL)


def _logits_body(q_ref, nk_ref, o_ref):
    o_ref[...] = jax.lax.dot_general(
        q_ref[...], nk_ref[...],
        dimension_numbers=(((1,), (1,)), ((), ())),
        preferred_element_type=jnp.float32,
    ) * 0.0625


def _logits(query, nk):
    return pl.pallas_call(
        _logits_body,
        grid=(N_INPUT // N_CHUNK,),
        in_specs=[pl.BlockSpec((B, D_ROUTING), lambda n: (0, 0)),
                  pl.BlockSpec((N_CHUNK, D_ROUTING), lambda n: (n, 0))],
        out_specs=pl.BlockSpec((B, N_CHUNK), lambda n: (0, n)),
        out_shape=jax.ShapeDtypeStruct((B, N_INPUT), jnp.float32),
    )(query, nk)


def kernel(x, W1, b1, ln_g, ln_b, W2, b2, neuron_keys, k_input):
    gc = _maxpool(x)
    h = gc @ W1 + b1
    h = jax.nn.gelu(h, approximate=False)
    mu = jnp.mean(h, axis=-1, keepdims=True)
    var = jnp.mean((h - mu) ** 2, axis=-1, keepdims=True)
    h = (h - mu) / jnp.sqrt(var + 1e-5) * ln_g + ln_b
    query = h @ W2 + b2
    logits = _logits(query, neuron_keys)
    _, input_idx = jax.lax.top_k(logits, K)
    rows = jnp.arange(B)[:, None]
    one_hot = jnp.zeros_like(logits).at[rows, input_idx].set(1.0)
    return (input_idx, one_hot)
